# Initial kernel scaffold; baseline (speedup 1.0000x reference)
import jax
import jax.numpy as jnp
from jax.experimental import pallas as pl


def kernel(h, edge_index, etypes, basis0, coeff0, cb0, w1_0, b1_0, g1_0, gb1_0, w2_0, b2_0, bng_0, bnb_0, basis1, coeff1, cb1, w1_1, b1_1, g1_1, gb1_1, w2_1, b2_1, bng_1, bnb_1):
    raise NotImplementedError("write your pallas kernel here")


# speedup vs baseline: 14.6477x; 14.6477x over previous
import functools

import jax
import jax.numpy as jnp
from jax import lax
from jax.experimental import pallas as pl
from jax.experimental.pallas import tpu as pltpu
from jax.experimental.pallas import tpu_sc as plsc

N, E, D, R = 10000, 320000, 128, 8
NC, NS = 2, 16
NW = NC * NS
CHUNK = 128
CPW = 79
EPAD = NW * CPW * CHUNK
NPAD = 10240
ROWS_PT = NPAD // NS
BN_EPS = 1e-5
NBLK = 2000
NB = N // NBLK


def _wcomb_body(coeff_ref, basis_ref, w_ref):
    w_ref[...] = jnp.dot(coeff_ref[...], basis_ref[...],
                         preferred_element_type=jnp.float32)


def _wcomb(coeff, basis):
    basis2 = basis.reshape(R, D * D)
    return pl.pallas_call(
        _wcomb_body,
        out_shape=jax.ShapeDtypeStruct((R, D * D), jnp.float32),
    )(coeff, basis2)


def _hall_body(x_ref, w_ref, out_ref):
    out_ref[0] = jnp.dot(x_ref[...], w_ref[0],
                         preferred_element_type=jnp.float32)


def _hall(x, w):
    return pl.pallas_call(
        _hall_body,
        grid=(NB, R),
        in_specs=[
            pl.BlockSpec((NBLK, D), lambda i, r: (i, 0)),
            pl.BlockSpec((1, D, D), lambda i, r: (r, 0, 0)),
        ],
        out_specs=pl.BlockSpec((1, NBLK, D), lambda i, r: (r, i, 0)),
        out_shape=jax.ShapeDtypeStruct((R, N, D), jnp.float32),
    )(x, w.reshape(R, D, D))


def _stage1_body(p_ref, cb_ref, w1_ref, b1_ref, hdn_ref, st_ref):
    i = pl.program_id(0)
    agg = p_ref[0] + p_ref[1] + cb_ref[...]
    hdn = jnp.dot(agg, w1_ref[...], preferred_element_type=jnp.float32)
    hdn = hdn + b1_ref[...]
    hdn_ref[...] = hdn
    s0 = jnp.sum(hdn, axis=0, keepdims=True)
    s1 = jnp.sum(hdn * hdn, axis=0, keepdims=True)
    st = jnp.concatenate([s0, s1], axis=0)

    @pl.when(i == 0)
    def _():
        st_ref[...] = jnp.zeros_like(st_ref)

    st_ref[...] += st


def _stage1(partials, cb, w1, b1):
    return pl.pallas_call(
        _stage1_body,
        grid=(NB,),
        in_specs=[
            pl.BlockSpec((NC, NBLK, D), lambda i: (0, i, 0)),
            pl.BlockSpec((D,), lambda i: (0,)),
            pl.BlockSpec((D, D), lambda i: (0, 0)),
            pl.BlockSpec((D,), lambda i: (0,)),
        ],
        out_specs=[
            pl.BlockSpec((NBLK, D), lambda i: (i, 0)),
            pl.BlockSpec((2, D), lambda i: (0, 0)),
        ],
        out_shape=[
            jax.ShapeDtypeStruct((N, D), jnp.float32),
            jax.ShapeDtypeStruct((2, D), jnp.float32),
        ],
    )(partials, cb, w1, b1)


def _stage2_body(hdn_ref, st_ref, g_ref, gb_ref, w2_ref, b2_ref,
                 y_ref, st2_ref):
    i = pl.program_id(0)
    mean = st_ref[0] * (1.0 / N)
    var = st_ref[1] * (1.0 / N) - mean * mean
    scale = lax.rsqrt(var + BN_EPS) * g_ref[...]
    shift = gb_ref[...] - mean * scale
    xb = jnp.maximum(hdn_ref[...] * scale + shift, 0.0)
    y = jnp.dot(xb, w2_ref[...], preferred_element_type=jnp.float32)
    y = y + b2_ref[...]
    y_ref[...] = y
    s0 = jnp.sum(y, axis=0, keepdims=True)
    s1 = jnp.sum(y * y, axis=0, keepdims=True)
    st = jnp.concatenate([s0, s1], axis=0)

    @pl.when(i == 0)
    def _():
        st2_ref[...] = jnp.zeros_like(st2_ref)

    st2_ref[...] += st


def _stage2(hdn, stats, g1, gb1, w2, b2):
    return pl.pallas_call(
        _stage2_body,
        grid=(NB,),
        in_specs=[
            pl.BlockSpec((NBLK, D), lambda i: (i, 0)),
            pl.BlockSpec((2, D), lambda i: (0, 0)),
            pl.BlockSpec((D,), lambda i: (0,)),
            pl.BlockSpec((D,), lambda i: (0,)),
            pl.BlockSpec((D, D), lambda i: (0, 0)),
            pl.BlockSpec((D,), lambda i: (0,)),
        ],
        out_specs=[
            pl.BlockSpec((NBLK, D), lambda i: (i, 0)),
            pl.BlockSpec((2, D), lambda i: (0, 0)),
        ],
        out_shape=[
            jax.ShapeDtypeStruct((N, D), jnp.float32),
            jax.ShapeDtypeStruct((2, D), jnp.float32),
        ],
    )(hdn, stats, g1, gb1, w2, b2)


def _stage3_body(y_ref, st_ref, g_ref, gb_ref, x_ref):
    mean = st_ref[0] * (1.0 / N)
    var = st_ref[1] * (1.0 / N) - mean * mean
    scale = lax.rsqrt(var + BN_EPS) * g_ref[...]
    shift = gb_ref[...] - mean * scale
    x_ref[...] = jnp.maximum(y_ref[...] * scale + shift, 0.0)


def _stage3(y, stats, bng, bnb):
    return pl.pallas_call(
        _stage3_body,
        grid=(NB,),
        in_specs=[
            pl.BlockSpec((NBLK, D), lambda i: (i, 0)),
            pl.BlockSpec((2, D), lambda i: (0, 0)),
            pl.BlockSpec((D,), lambda i: (0,)),
            pl.BlockSpec((D,), lambda i: (0,)),
        ],
        out_specs=pl.BlockSpec((NBLK, D), lambda i: (i, 0)),
        out_shape=jax.ShapeDtypeStruct((N, D), jnp.float32),
    )(y, stats, bng, bnb)


def _edge_agg_body(hall_ref, gidx_ref, dst_ref, zeros_ref, out_ref,
                   gidx_v, dst_v, rows_v, sem, agg):
    c = lax.axis_index("c")
    s = lax.axis_index("s")
    wid = s * NC + c
    pltpu.sync_copy(zeros_ref.at[pl.ds(s * ROWS_PT, ROWS_PT)],
                    agg.at[pl.ds(s * ROWS_PT, ROWS_PT)])
    pltpu.sync_copy(gidx_ref.at[wid], gidx_v)
    pltpu.sync_copy(dst_ref.at[wid], dst_v)
    plsc.subcore_barrier()

    def step(j, carry):
        pltpu.async_copy(hall_ref.at[gidx_v.at[j]], rows_v, sem).wait()
        pltpu.sync_copy(rows_v, agg.at[dst_v.at[j]], add=True)
        return carry

    lax.fori_loop(0, CPW, step, 0)
    plsc.subcore_barrier()
    pltpu.sync_copy(agg.at[pl.ds(s * ROWS_PT, ROWS_PT)],
                    out_ref.at[c, pl.ds(s * ROWS_PT, ROWS_PT)])


@functools.lru_cache(maxsize=None)
def _build_edge_agg():
    mesh = plsc.VectorSubcoreMesh(core_axis_name="c", subcore_axis_name="s")
    return pl.kernel(
        _edge_agg_body,
        out_type=jax.ShapeDtypeStruct((NC, NPAD, D), jnp.float32),
        mesh=mesh,
        scratch_types=[
            pltpu.VMEM((CPW, CHUNK), jnp.int32),
            pltpu.VMEM((CPW, CHUNK), jnp.int32),
            pltpu.VMEM((CHUNK, D), jnp.float32),
            pltpu.SemaphoreType.DMA,
            pltpu.VMEM_SHARED((NPAD, D), jnp.float32),
        ],
    )


def _layer(x, gidx, dst, zeros, basis, coeff, cb, w1, b1, g1, gb1,
           w2, b2, bng, bnb):
    w = _wcomb(coeff, basis)
    hall = _hall(x, w)
    partials = _build_edge_agg()(hall.reshape(R * N, D), gidx, dst, zeros)
    hdn, st1 = _stage1(partials, cb, w1, b1)
    y, st2 = _stage2(hdn, st1, g1, gb1, w2, b2)
    return _stage3(y, st2, bng, bnb)


def kernel(h, edge_index, etypes,
           basis0, coeff0, cb0, w1_0, b1_0, g1_0, gb1_0, w2_0, b2_0,
           bng_0, bnb_0,
           basis1, coeff1, cb1, w1_1, b1_1, g1_1, gb1_1, w2_1, b2_1,
           bng_1, bnb_1):
    src, dst = edge_index[0], edge_index[1]
    gidx = etypes * N + src
    pad = EPAD - E
    gidx = jnp.concatenate([gidx, jnp.zeros((pad,), jnp.int32)])
    dstp = jnp.concatenate([dst, jnp.full((pad,), N, jnp.int32)])
    gidx = gidx.reshape(NW, CPW, CHUNK)
    dstp = dstp.reshape(NW, CPW, CHUNK)
    zeros = jnp.zeros((NPAD, D), jnp.float32)

    x1 = _layer(h, gidx, dstp, zeros, basis0, coeff0, cb0, w1_0, b1_0,
                g1_0, gb1_0, w2_0, b2_0, bng_0, bnb_0)
    x2 = _layer(x1, gidx, dstp, zeros, basis1, coeff1, cb1, w1_1, b1_1,
                g1_1, gb1_1, w2_1, b2_1, bng_1, bnb_1)
    return jnp.stack([h, x1, x2])
